# baseline (speedup 1.0000x reference)
import functools

import jax
import jax.numpy as jnp
from jax import lax
from jax.experimental import pallas as pl
from jax.experimental.pallas import tpu as pltpu
from jax.experimental.pallas import tpu_sc as plsc

HID = 768
EPS = 1e-12
T2 = 512


def _make_sc_gather(n_tokens: int):
    info = plsc.get_sparse_core_info()
    nc, ns = info.num_cores, info.num_subcores
    nw = nc * ns
    b_per_w = n_tokens // nw
    chunk = 64
    n_chunks = b_per_w // chunk
    mesh = plsc.VectorSubcoreMesh(core_axis_name="c", subcore_axis_name="s")

    def _compress_pairs(rows_v):
        def body(t2, _):
            for j in range(HID // 16):
                sl = pl.ds(16 * j, 16)
                lo = rows_v[2 * t2, sl]
                hi = rows_v[2 * t2 + 1, sl]
                rows_v[t2, sl] = lax.bitwise_or(
                    lax.bitwise_and(hi, jnp.int32(-65536)),
                    lax.shift_right_logical(lo, 16))
            return 0

        lax.fori_loop(0, chunk // 2, body, 0)

    @functools.partial(
        pl.kernel,
        mesh=mesh,
        out_type=jax.ShapeDtypeStruct((n_tokens // 2, HID), jnp.int32),
        scratch_types=[
            pltpu.VMEM((b_per_w,), jnp.int32),
            pltpu.VMEM((chunk, HID), jnp.int32),
            pltpu.VMEM((chunk, HID), jnp.int32),
            pltpu.SemaphoreType.DMA,
            pltpu.SemaphoreType.DMA,
            pltpu.SemaphoreType.DMA,
            pltpu.SemaphoreType.DMA,
        ],
    )
    def gather_k(idx_hbm, table_hbm, out_hbm, idx_v, rows0, rows1,
                 si0, si1, so0, so1):
        wid = lax.axis_index("s") * nc + lax.axis_index("c")
        base = wid * b_per_w
        pltpu.sync_copy(idx_hbm.at[pl.ds(base, b_per_w)], idx_v)
        bufs = (rows0, rows1)
        sin = (si0, si1)
        sout = (so0, so1)
        ins = [None] * n_chunks
        outs = [None] * n_chunks
        ins[0] = pltpu.async_copy(
            table_hbm.at[idx_v.at[pl.ds(0, chunk)]], rows0, si0)
        for c in range(n_chunks):
            if c + 1 < n_chunks:
                if c >= 1:
                    outs[c - 1].wait()
                ins[c + 1] = pltpu.async_copy(
                    table_hbm.at[idx_v.at[pl.ds((c + 1) * chunk, chunk)]],
                    bufs[(c + 1) % 2], sin[(c + 1) % 2])
            ins[c].wait()
            buf = bufs[c % 2]
            _compress_pairs(buf)
            outs[c] = pltpu.async_copy(
                buf.at[pl.ds(0, chunk // 2)],
                out_hbm.at[pl.ds(
                    pl.multiple_of((base + c * chunk) // 2, chunk // 2),
                    chunk // 2)],
                sout[c % 2])
        if n_chunks >= 2:
            outs[n_chunks - 2].wait()
        outs[n_chunks - 1].wait()

    return gather_k


def _ln_body(w_ref, p_ref, ttf_ref, te_ref, g_ref, b_ref, o_ref):
    u = lax.bitcast_convert_type(w_ref[...], jnp.int32)
    w_lo = lax.bitcast_convert_type(lax.shift_left(u, 16), jnp.float32)
    w_hi = lax.bitcast_convert_type(
        lax.bitwise_and(u, jnp.int32(-65536)), jnp.float32)
    t0 = te_ref[0, :][None, :]
    t1 = te_ref[1, :][None, :]
    td = t1 - t0
    g = g_ref[...]
    b = b_ref[...]

    def ln(w, p, ttf):
        e = w + p + t0 + ttf * td
        mu = jnp.mean(e, axis=1, keepdims=True)
        ex2 = jnp.mean(e * e, axis=1, keepdims=True)
        var = ex2 - mu * mu
        return (e - mu) * lax.rsqrt(var + EPS) * g + b

    o_ref[0, :, 0, :] = ln(w_lo, p_ref[:, 0, :], ttf_ref[0, 0, :, 0][:, None])
    o_ref[0, :, 1, :] = ln(w_hi, p_ref[:, 1, :], ttf_ref[0, 0, :, 1][:, None])


def kernel(input_ids, token_type_ids, word_emb, pos_emb, type_emb, ln_gamma,
           ln_beta):
    B, S = input_ids.shape
    n = B * S
    spt2 = (S // 2) // T2

    ids = input_ids.astype(jnp.int32).reshape(n)
    table_i32 = lax.bitcast_convert_type(word_emb, jnp.int32)
    words = _make_sc_gather(n)(ids, table_i32)

    pos4 = pos_emb.reshape(S // 2, 2, HID)
    ttf4 = token_type_ids.astype(jnp.float32).reshape(B * spt2, 1, T2, 2)

    out = pl.pallas_call(
        _ln_body,
        grid=(spt2, B),
        in_specs=[
            pl.BlockSpec((T2, HID), lambda s, b: (b * spt2 + s, 0)),
            pl.BlockSpec((T2, 2, HID), lambda s, b: (s, 0, 0)),
            pl.BlockSpec((1, 1, T2, 2), lambda s, b: (b * spt2 + s, 0, 0, 0)),
            pl.BlockSpec((2, HID), lambda s, b: (0, 0)),
            pl.BlockSpec((1, HID), lambda s, b: (0, 0)),
            pl.BlockSpec((1, HID), lambda s, b: (0, 0)),
        ],
        out_specs=pl.BlockSpec((1, T2, 2, HID), lambda s, b: (b, s, 0, 0)),
        out_shape=jax.ShapeDtypeStruct((B, S // 2, 2, HID), jnp.float32),
    )(words, pos4, ttf4, type_emb,
      ln_gamma.reshape(1, HID), ln_beta.reshape(1, HID))
    return out.reshape(B, S, HID)


# speedup vs baseline: 5.6011x; 5.6011x over previous
import functools

import jax
import jax.numpy as jnp
from jax import lax
from jax.experimental import pallas as pl
from jax.experimental.pallas import tpu as pltpu
from jax.experimental.pallas import tpu_sc as plsc

HID = 768
EPS = 1e-12
T = 1024


def _make_sc_gather(n_tokens: int):
    info = plsc.get_sparse_core_info()
    nc, ns = info.num_cores, info.num_subcores
    nw = nc * ns
    b_per_w = n_tokens // nw
    chunk = 64
    n_chunks = b_per_w // chunk
    mesh = plsc.VectorSubcoreMesh(core_axis_name="c", subcore_axis_name="s")

    @functools.partial(
        pl.kernel,
        mesh=mesh,
        out_type=jax.ShapeDtypeStruct((n_tokens, HID), jnp.float32),
        scratch_types=[
            pltpu.VMEM((b_per_w,), jnp.int32),
            pltpu.VMEM((chunk, HID), jnp.float32),
            pltpu.VMEM((chunk, HID), jnp.float32),
            pltpu.SemaphoreType.DMA,
            pltpu.SemaphoreType.DMA,
            pltpu.SemaphoreType.DMA,
            pltpu.SemaphoreType.DMA,
        ],
    )
    def gather_k(idx_hbm, table_hbm, out_hbm, idx_v, rows0, rows1,
                 si0, si1, so0, so1):
        wid = lax.axis_index("s") * nc + lax.axis_index("c")
        base = wid * b_per_w
        pltpu.sync_copy(idx_hbm.at[pl.ds(base, b_per_w)], idx_v)
        bufs = (rows0, rows1)
        sin = (si0, si1)
        sout = (so0, so1)
        ins = [None] * n_chunks
        outs = [None] * n_chunks
        ins[0] = pltpu.async_copy(
            table_hbm.at[idx_v.at[pl.ds(0, chunk)]], rows0, si0)
        for c in range(n_chunks):
            if c + 1 < n_chunks:
                if c >= 1:
                    outs[c - 1].wait()
                ins[c + 1] = pltpu.async_copy(
                    table_hbm.at[idx_v.at[pl.ds((c + 1) * chunk, chunk)]],
                    bufs[(c + 1) % 2], sin[(c + 1) % 2])
            ins[c].wait()
            outs[c] = pltpu.async_copy(
                bufs[c % 2], out_hbm.at[pl.ds(base + c * chunk, chunk)],
                sout[c % 2])
        if n_chunks >= 2:
            outs[n_chunks - 2].wait()
        outs[n_chunks - 1].wait()

    return gather_k


def _ln_body(w_ref, p_ref, ttf_ref, te_ref, g_ref, b_ref, o_ref):
    w = w_ref[...]
    p = p_ref[...]
    ttf = ttf_ref[0, 0, :][:, None]
    t0 = te_ref[0, :][None, :]
    t1 = te_ref[1, :][None, :]
    e = w + p + t0 + ttf * (t1 - t0)
    mu = jnp.mean(e, axis=1, keepdims=True)
    ex2 = jnp.mean(e * e, axis=1, keepdims=True)
    var = ex2 - mu * mu
    o_ref[0] = (e - mu) * lax.rsqrt(var + EPS) * g_ref[...] + b_ref[...]


def kernel(input_ids, token_type_ids, word_emb, pos_emb, type_emb, ln_gamma,
           ln_beta):
    B, S = input_ids.shape
    n = B * S
    spt = S // T

    ids = input_ids.astype(jnp.int32).reshape(n)
    words = _make_sc_gather(n)(ids, word_emb)

    ttf = token_type_ids.astype(jnp.float32).reshape(n // T, 1, T)

    out = pl.pallas_call(
        _ln_body,
        grid=(spt, B),
        in_specs=[
            pl.BlockSpec((T, HID), lambda s, b: (b * spt + s, 0)),
            pl.BlockSpec((T, HID), lambda s, b: (s, 0)),
            pl.BlockSpec((1, 1, T), lambda s, b: (b * spt + s, 0, 0)),
            pl.BlockSpec((2, HID), lambda s, b: (0, 0)),
            pl.BlockSpec((1, HID), lambda s, b: (0, 0)),
            pl.BlockSpec((1, HID), lambda s, b: (0, 0)),
        ],
        out_specs=pl.BlockSpec((1, T, HID), lambda s, b: (b, s, 0)),
        out_shape=jax.ShapeDtypeStruct((B, S, HID), jnp.float32),
    )(words, pos_emb, ttf, type_emb,
      ln_gamma.reshape(1, HID), ln_beta.reshape(1, HID))
    return out


# speedup vs baseline: 5.6474x; 1.0083x over previous
import functools

import jax
import jax.numpy as jnp
from jax import lax
from jax.experimental import pallas as pl
from jax.experimental.pallas import tpu as pltpu
from jax.experimental.pallas import tpu_sc as plsc

HID = 768
EPS = 1e-12
T = 2048


def _make_sc_gather(n_tokens: int):
    info = plsc.get_sparse_core_info()
    nc, ns = info.num_cores, info.num_subcores
    nw = nc * ns
    b_per_w = n_tokens // nw
    chunk = 64
    n_chunks = b_per_w // chunk
    mesh = plsc.VectorSubcoreMesh(core_axis_name="c", subcore_axis_name="s")

    @functools.partial(
        pl.kernel,
        mesh=mesh,
        out_type=jax.ShapeDtypeStruct((n_tokens, HID), jnp.float32),
        scratch_types=[
            pltpu.VMEM((b_per_w,), jnp.int32),
            pltpu.VMEM((chunk, HID), jnp.float32),
            pltpu.VMEM((chunk, HID), jnp.float32),
            pltpu.SemaphoreType.DMA,
            pltpu.SemaphoreType.DMA,
            pltpu.SemaphoreType.DMA,
            pltpu.SemaphoreType.DMA,
        ],
    )
    def gather_k(idx_hbm, table_hbm, out_hbm, idx_v, rows0, rows1,
                 si0, si1, so0, so1):
        wid = lax.axis_index("s") * nc + lax.axis_index("c")
        base = wid * b_per_w
        pltpu.sync_copy(idx_hbm.at[pl.ds(base, b_per_w)], idx_v)
        bufs = (rows0, rows1)
        sin = (si0, si1)
        sout = (so0, so1)
        ins = [None] * n_chunks
        outs = [None] * n_chunks
        ins[0] = pltpu.async_copy(
            table_hbm.at[idx_v.at[pl.ds(0, chunk)]], rows0, si0)
        for c in range(n_chunks):
            if c + 1 < n_chunks:
                if c >= 1:
                    outs[c - 1].wait()
                ins[c + 1] = pltpu.async_copy(
                    table_hbm.at[idx_v.at[pl.ds((c + 1) * chunk, chunk)]],
                    bufs[(c + 1) % 2], sin[(c + 1) % 2])
            ins[c].wait()
            outs[c] = pltpu.async_copy(
                bufs[c % 2], out_hbm.at[pl.ds(base + c * chunk, chunk)],
                sout[c % 2])
        if n_chunks >= 2:
            outs[n_chunks - 2].wait()
        outs[n_chunks - 1].wait()

    return gather_k


def _ln_body(w_ref, p_ref, ttf_ref, te_ref, g_ref, b_ref, o_ref):
    w = w_ref[...]
    p = p_ref[...]
    ttf = ttf_ref[0, 0, :][:, None]
    t0 = te_ref[0, :][None, :]
    t1 = te_ref[1, :][None, :]
    e = w + p + t0 + ttf * (t1 - t0)
    mu = jnp.mean(e, axis=1, keepdims=True)
    ex2 = jnp.mean(e * e, axis=1, keepdims=True)
    var = ex2 - mu * mu
    o_ref[0] = (e - mu) * lax.rsqrt(var + EPS) * g_ref[...] + b_ref[...]


def kernel(input_ids, token_type_ids, word_emb, pos_emb, type_emb, ln_gamma,
           ln_beta):
    B, S = input_ids.shape
    n = B * S
    spt = S // T

    ids = input_ids.astype(jnp.int32).reshape(n)
    words = _make_sc_gather(n)(ids, word_emb)

    ttf = token_type_ids.astype(jnp.float32).reshape(n // T, 1, T)

    out = pl.pallas_call(
        _ln_body,
        grid=(spt, B),
        in_specs=[
            pl.BlockSpec((T, HID), lambda s, b: (b * spt + s, 0)),
            pl.BlockSpec((T, HID), lambda s, b: (s, 0)),
            pl.BlockSpec((1, 1, T), lambda s, b: (b * spt + s, 0, 0)),
            pl.BlockSpec((2, HID), lambda s, b: (0, 0)),
            pl.BlockSpec((1, HID), lambda s, b: (0, 0)),
            pl.BlockSpec((1, HID), lambda s, b: (0, 0)),
        ],
        out_specs=pl.BlockSpec((1, T, HID), lambda s, b: (b, s, 0)),
        out_shape=jax.ShapeDtypeStruct((B, S, HID), jnp.float32),
    )(words, pos_emb, ttf, type_emb,
      ln_gamma.reshape(1, HID), ln_beta.reshape(1, HID))
    return out
